# baseline (speedup 1.0000x reference)
import jax
import jax.numpy as jnp
from jax.experimental import pallas as pl
from jax.experimental.pallas import tpu as pltpu

_EPS = 1e-09
_W = 5000
_BLOCK_R = 256
_THIRD = 1.0 / 3.0


def _body(x_ref, o_ref):
    x = x_ref[...]
    lane = jax.lax.broadcasted_iota(jnp.int32, (1, _W), 1)
    valid = lane < (_W - 2)
    not_first = lane >= 1
    not_last2 = lane < (_W - 1)
    pos_inf = jnp.float32(jnp.inf)
    neg_inf = jnp.float32(-jnp.inf)
    for _ in range(4):
        m = jnp.minimum(
            jnp.minimum(x, pltpu.roll(x, _W - 1, 1)), pltpu.roll(x, _W - 2, 1)
        )
        pmin = jnp.min(jnp.where(valid, m, pos_inf), axis=1, keepdims=True)
        pmax = jnp.max(jnp.where(valid, m, neg_inf), axis=1, keepdims=True)
        inv = 1.0 / (pmax - pmin)
        m = jnp.where(valid, (m - pmin) * inv + _EPS, 0.0)
        left = jnp.where(not_first, pltpu.roll(m, 1, 1), 0.0)
        right = jnp.where(not_last2, pltpu.roll(m, _W - 1, 1), 0.0)
        x = (left + m + right) * _THIRD
    o_ref[...] = x


def kernel(x):
    x = x.reshape(-1, _W).astype(jnp.float32)
    n = x.shape[0]
    out = pl.pallas_call(
        _body,
        grid=(n // _BLOCK_R,),
        in_specs=[pl.BlockSpec((_BLOCK_R, _W), lambda i: (i, 0))],
        out_specs=pl.BlockSpec((_BLOCK_R, _W), lambda i: (i, 0)),
        out_shape=jax.ShapeDtypeStruct((n, _W), jnp.float32),
        compiler_params=pltpu.CompilerParams(
            dimension_semantics=("parallel",)
        ),
    )(x)
    return out.reshape(-1, 1, 50, 100)


# speedup vs baseline: 1.1391x; 1.1391x over previous
import jax
import jax.numpy as jnp
from jax.experimental import pallas as pl
from jax.experimental.pallas import tpu as pltpu

_EPS = 1e-09
_W = 5000
_BLOCK_R = 256
_THIRD = 1.0 / 3.0


def _body(x_ref, o_ref):
    x = x_ref[...]
    r = x.shape[0]
    zero1 = jnp.zeros((r, 1), x.dtype)
    zero2 = jnp.zeros((r, 2), x.dtype)
    for _ in range(4):
        m = jnp.minimum(jnp.minimum(x[:, :-2], x[:, 1:-1]), x[:, 2:])
        pmin = jnp.min(m, axis=1, keepdims=True)
        pmax = jnp.max(m, axis=1, keepdims=True)
        inv = 1.0 / (pmax - pmin)
        m = (m - pmin) * inv + _EPS
        m = jnp.concatenate([m, zero2], axis=1)
        left = jnp.concatenate([zero1, m[:, :-1]], axis=1)
        right = jnp.concatenate([m[:, 1:], zero1], axis=1)
        x = (left + m + right) * _THIRD
    o_ref[...] = x


def kernel(x):
    x = x.reshape(-1, _W).astype(jnp.float32)
    n = x.shape[0]
    out = pl.pallas_call(
        _body,
        grid=(n // _BLOCK_R,),
        in_specs=[pl.BlockSpec((_BLOCK_R, _W), lambda i: (i, 0))],
        out_specs=pl.BlockSpec((_BLOCK_R, _W), lambda i: (i, 0)),
        out_shape=jax.ShapeDtypeStruct((n, _W), jnp.float32),
        compiler_params=pltpu.CompilerParams(
            dimension_semantics=("parallel",)
        ),
    )(x)
    return out.reshape(-1, 1, 50, 100)
